# baseline (speedup 1.0000x reference)
import functools

import jax
import jax.numpy as jnp
from jax import lax
from jax.experimental import pallas as pl
from jax.experimental.pallas import tpu as pltpu
from jax.experimental.pallas import tpu_sc as plsc

B = 4096
D = 64
HID = 128
POSTER_DIM = 2048

_NC, _NS = 2, 16
NW = _NC * _NS
BPW = B // NW


@functools.cache
def _make_gather3():
    mesh = plsc.VectorSubcoreMesh(
        core_axis_name="c", subcore_axis_name="s", num_cores=_NC)

    @functools.partial(
        pl.kernel,
        mesh=mesh,
        compiler_params=pltpu.CompilerParams(use_tc_tiling_on_sc=False),
        out_type=[
            jax.ShapeDtypeStruct((B, D), jnp.float32),
            jax.ShapeDtypeStruct((B, D), jnp.float32),
            jax.ShapeDtypeStruct((B, D), jnp.float32),
        ],
        scratch_types=[
            pltpu.VMEM((BPW,), jnp.int32),
            pltpu.VMEM((BPW,), jnp.int32),
            pltpu.VMEM((BPW, D), jnp.float32),
            pltpu.VMEM((BPW, D), jnp.float32),
            pltpu.VMEM((BPW, D), jnp.float32),
            pltpu.SemaphoreType.DMA,
        ],
    )
    def _gather3(uidx_hbm, iidx_hbm, umf_hbm, imf_hbm, uv_hbm,
                 out_umf, out_imf, out_uv,
                 uidx_v, iidx_v, umf_v, imf_v, uv_v, sem):
        wid = lax.axis_index("s") * _NC + lax.axis_index("c")
        base = wid * BPW
        pltpu.sync_copy(uidx_hbm.at[pl.ds(base, BPW)], uidx_v)
        pltpu.sync_copy(iidx_hbm.at[pl.ds(base, BPW)], iidx_v)
        cp1 = pltpu.async_copy(umf_hbm.at[uidx_v], umf_v, sem)
        cp2 = pltpu.async_copy(imf_hbm.at[iidx_v], imf_v, sem)
        cp3 = pltpu.async_copy(uv_hbm.at[uidx_v], uv_v, sem)
        cp1.wait()
        cp2.wait()
        cp3.wait()
        pltpu.sync_copy(umf_v, out_umf.at[pl.ds(base, BPW)])
        pltpu.sync_copy(imf_v, out_imf.at[pl.ds(base, BPW)])
        pltpu.sync_copy(uv_v, out_uv.at[pl.ds(base, BPW)])

    return _gather3


BB = 512


def _mlp_body(poster_ref, w1_ref, b1_ref, w2_ref, b2_ref, iv_ref):
    h = jnp.dot(poster_ref[...], w1_ref[...],
                preferred_element_type=jnp.float32) + b1_ref[...]
    h = jnp.maximum(h, 0.0)
    iv_ref[...] = jnp.dot(h, w2_ref[...],
                          preferred_element_type=jnp.float32) + b2_ref[...]


def _mlp(poster, w1, b1, w2, b2):
    return pl.pallas_call(
        _mlp_body,
        grid=(B // BB,),
        in_specs=[
            pl.BlockSpec((BB, POSTER_DIM), lambda i: (i, 0)),
            pl.BlockSpec((POSTER_DIM, HID), lambda i: (0, 0)),
            pl.BlockSpec((1, HID), lambda i: (0, 0)),
            pl.BlockSpec((HID, D), lambda i: (0, 0)),
            pl.BlockSpec((1, D), lambda i: (0, 0)),
        ],
        out_specs=pl.BlockSpec((BB, D), lambda i: (i, 0)),
        out_shape=jax.ShapeDtypeStruct((B, D), jnp.float32),
    )(poster, w1, b1, w2, b2)


def _combine_body(umf_ref, imf_ref, uv_ref, iv_ref, wo_ref, bo_ref, out_ref):
    vec = umf_ref[...] * imf_ref[...] + uv_ref[...] * iv_ref[...]
    logits = jnp.sum(vec * wo_ref[...], axis=1, keepdims=True) + bo_ref[...]
    out_ref[...] = jax.nn.sigmoid(logits)


def _combine(umf, imf, uv, iv, wo_t, bo):
    return pl.pallas_call(
        _combine_body,
        grid=(1,),
        in_specs=[
            pl.BlockSpec((B, D), lambda i: (0, 0)),
            pl.BlockSpec((B, D), lambda i: (0, 0)),
            pl.BlockSpec((B, D), lambda i: (0, 0)),
            pl.BlockSpec((B, D), lambda i: (0, 0)),
            pl.BlockSpec((1, D), lambda i: (0, 0)),
            pl.BlockSpec((1, 1), lambda i: (0, 0)),
        ],
        out_specs=pl.BlockSpec((B, 1), lambda i: (0, 0)),
        out_shape=jax.ShapeDtypeStruct((B, 1), jnp.float32),
    )(umf, imf, uv, iv, wo_t, bo)


def kernel(user_indices, item_indices, poster_embeddings, U_mf, I_mf, U_v,
           W1, b1, W2, b2, Wo, bo):
    ui = user_indices.astype(jnp.int32)
    ii = item_indices.astype(jnp.int32)
    umf, imf, uv = _make_gather3()(ui, ii, U_mf, I_mf, U_v)
    iv = _mlp(poster_embeddings, W1, b1.reshape(1, HID), W2, b2.reshape(1, D))
    rating = _combine(umf, imf, uv, iv, Wo.reshape(1, D), bo.reshape(1, 1))
    return rating


# speedup vs baseline: 1.3008x; 1.3008x over previous
import functools

import jax
import jax.numpy as jnp
from jax import lax
from jax.experimental import pallas as pl
from jax.experimental.pallas import tpu as pltpu
from jax.experimental.pallas import tpu_sc as plsc

B = 4096
D = 64
HID = 128
POSTER_DIM = 2048
NROW = 100000
TILE_H = 8
NTILE = NROW // TILE_H

_NC, _NS = 2, 16
NW = _NC * _NS
BPW = B // NW
CHUNK = 64
L = 16


@functools.cache
def _make_gather3():
    mesh = plsc.VectorSubcoreMesh(
        core_axis_name="c", subcore_axis_name="s", num_cores=_NC)

    @functools.partial(
        pl.kernel,
        mesh=mesh,
        out_type=[
            jax.ShapeDtypeStruct((B, D), jnp.float32),
            jax.ShapeDtypeStruct((B, D), jnp.float32),
            jax.ShapeDtypeStruct((B, D), jnp.float32),
        ],
        scratch_types=[
            pltpu.VMEM((BPW,), jnp.int32),
            pltpu.VMEM((BPW,), jnp.int32),
            pltpu.VMEM((BPW, D), jnp.float32),
            pltpu.SemaphoreType.DMA,
        ],
    )
    def _gather3(ui_hbm, ii_hbm,
                 umf_hbm, imf_hbm, uv_hbm,
                 out_umf, out_imf, out_uv,
                 ui_v, ii_v, rows_v, sem):
        wid = lax.axis_index("s") * _NC + lax.axis_index("c")
        base = wid * BPW
        pltpu.sync_copy(ui_hbm.at[pl.ds(base, BPW)], ui_v)
        pltpu.sync_copy(ii_hbm.at[pl.ds(base, BPW)], ii_v)

        def one_table(table_hbm, idx_v, out_hbm):
            def group_body(g, _):
                j0 = g * L
                grp = idx_v[pl.ds(j0, L)]
                for k in range(L):
                    pltpu.async_copy(
                        table_hbm.at[grp[k]],
                        rows_v.at[j0 + k], sem)

                def drain(k, _):
                    pltpu.make_async_copy(
                        table_hbm.at[0], rows_v.at[0], sem).wait()
                    return 0

                lax.fori_loop(0, L, drain, 0)
                return 0

            lax.fori_loop(0, BPW // L, group_body, 0)
            pltpu.sync_copy(rows_v, out_hbm.at[pl.ds(base, BPW)])

        one_table(umf_hbm, ui_v, out_umf)
        one_table(imf_hbm, ii_v, out_imf)
        one_table(uv_hbm, ui_v, out_uv)

    return _gather3


BB = 512


def _mlp_body(poster_ref, w1_ref, b1_ref, w2_ref, b2_ref, wo_ref, bo_ref,
              umf_ref, imf_ref, uv_ref, out_ref):
    h = jnp.dot(poster_ref[...], w1_ref[...],
                preferred_element_type=jnp.float32) + b1_ref[...]
    h = jnp.maximum(h, 0.0)
    iv = jnp.dot(h, w2_ref[...], preferred_element_type=jnp.float32) + b2_ref[...]
    vec = umf_ref[...] * imf_ref[...] + uv_ref[...] * iv
    logits = jnp.sum(vec * wo_ref[...], axis=1, keepdims=True) + bo_ref[...]
    out_ref[...] = jax.nn.sigmoid(logits)


def _mlp_combine(poster, w1, b1, w2, b2, wo_t, bo, umf, imf, uv):
    return pl.pallas_call(
        _mlp_body,
        grid=(B // BB,),
        in_specs=[
            pl.BlockSpec((BB, POSTER_DIM), lambda i: (i, 0)),
            pl.BlockSpec((POSTER_DIM, HID), lambda i: (0, 0)),
            pl.BlockSpec((1, HID), lambda i: (0, 0)),
            pl.BlockSpec((HID, D), lambda i: (0, 0)),
            pl.BlockSpec((1, D), lambda i: (0, 0)),
            pl.BlockSpec((1, D), lambda i: (0, 0)),
            pl.BlockSpec((1, 1), lambda i: (0, 0)),
            pl.BlockSpec((BB, D), lambda i: (i, 0)),
            pl.BlockSpec((BB, D), lambda i: (i, 0)),
            pl.BlockSpec((BB, D), lambda i: (i, 0)),
        ],
        out_specs=pl.BlockSpec((BB, 1), lambda i: (i, 0)),
        out_shape=jax.ShapeDtypeStruct((B, 1), jnp.float32),
    )(poster, w1, b1, w2, b2, wo_t, bo, umf, imf, uv)


def kernel(user_indices, item_indices, poster_embeddings, U_mf, I_mf, U_v,
           W1, b1, W2, b2, Wo, bo):
    ui = user_indices.astype(jnp.int32)
    ii = item_indices.astype(jnp.int32)
    umf, imf, uv = _make_gather3()(ui, ii, U_mf, I_mf, U_v)
    rating = _mlp_combine(
        poster_embeddings, W1,
        b1.reshape(1, HID), W2, b2.reshape(1, D),
        Wo.reshape(1, D), bo.reshape(1, 1),
        umf, imf, uv)
    return rating
